# baseline (speedup 1.0000x reference)
import functools

import jax
import jax.numpy as jnp
from jax.experimental import pallas as pl
from jax.experimental.pallas import tpu as pltpu

B, S, D_IN, D_OUT, E = 4, 2048, 1024, 1024, 64

BS = 512
BO = 1024


def _argmax_kernel(probs_ref, idx_ref):
    probs = probs_ref[...]
    idx_ref[...] = jnp.argmax(probs, axis=-1).astype(jnp.int32)[:, None]


def _moe_matmul_kernel(idx_ref, x_ref, w_ref, b_ref, out_ref):
    x = x_ref[0]
    w = w_ref[0]
    acc = jax.lax.dot_general(
        x, w,
        dimension_numbers=(((1,), (1,)), ((), ())),
        preferred_element_type=jnp.float32,
    )
    out_ref[0] = acc + b_ref[0]


def kernel(x, expert_probs, expert_weights, expert_biases):
    idx2d = pl.pallas_call(
        _argmax_kernel,
        out_shape=jax.ShapeDtypeStruct((B, 1), jnp.int32),
    )(expert_probs)
    expert_idx = idx2d.reshape(B)

    grid = (B, S // BS, D_OUT // BO)
    x_out = pl.pallas_call(
        _moe_matmul_kernel,
        grid_spec=pltpu.PrefetchScalarGridSpec(
            num_scalar_prefetch=1,
            grid=grid,
            in_specs=[
                pl.BlockSpec((1, BS, D_IN), lambda b, i, j, idx: (b, i, 0)),
                pl.BlockSpec((1, BO, D_IN), lambda b, i, j, idx: (idx[b], j, 0)),
                pl.BlockSpec((1, 1, BO), lambda b, i, j, idx: (idx[b], 0, j)),
            ],
            out_specs=pl.BlockSpec((1, BS, BO), lambda b, i, j, idx: (b, i, j)),
        ),
        out_shape=jax.ShapeDtypeStruct((B, S, D_OUT), jnp.float32),
    )(expert_idx, x, expert_weights, expert_biases.reshape(E, 1, D_OUT))

    return (x_out, expert_idx)


# speedup vs baseline: 1.2877x; 1.2877x over previous
import jax
import jax.numpy as jnp
from jax import lax
from jax.experimental import pallas as pl
from jax.experimental.pallas import tpu as pltpu

B, S, D_IN, D_OUT, E = 4, 2048, 1024, 1024, 64


def _moe_kernel(probs_ref, x_hbm, w_hbm, bias_ref,
                out_hbm, idx_ref,
                xv, wv, ov, idx_s,
                sem_x, sem_w, sem_o):
    def cp_x(b, slot):
        return pltpu.make_async_copy(x_hbm.at[b], xv.at[slot], sem_x.at[slot])

    def cp_w(b, slot):
        return pltpu.make_async_copy(w_hbm.at[idx_s[b]], wv.at[slot],
                                     sem_w.at[slot])

    def cp_o(b, slot):
        return pltpu.make_async_copy(ov.at[slot], out_hbm.at[b],
                                     sem_o.at[slot])

    cp_x(0, 0).start()
    cp_x(1, 1).start()

    for b in range(B):
        def body(e, carry):
            best_v, best_i = carry
            v = probs_ref[b, e]
            better = v > best_v
            return (jnp.where(better, v, best_v),
                    jnp.where(better, e, best_i))
        _, best_i = lax.fori_loop(0, E, body,
                                  (probs_ref[b, 0], jnp.int32(0)))
        idx_s[b] = best_i
        idx_ref[b] = best_i

    cp_w(0, 0).start()
    cp_w(1, 1).start()

    for b in range(B):
        slot = b % 2
        cp_x(b, slot).wait()
        cp_w(b, slot).wait()
        if b >= 2:
            cp_o(b - 2, slot).wait()
        acc = lax.dot_general(
            xv[slot], wv[slot],
            dimension_numbers=(((1,), (1,)), ((), ())),
            preferred_element_type=jnp.float32,
        )
        ov[slot] = acc + bias_ref[pl.ds(idx_s[b], 1), :]
        cp_o(b, slot).start()
        if b + 2 < B:
            cp_x(b + 2, slot).start()
            cp_w(b + 2, slot).start()

    cp_o(B - 2, 0).wait()
    cp_o(B - 1, 1).wait()


def kernel(x, expert_probs, expert_weights, expert_biases):
    x_out, expert_idx = pl.pallas_call(
        _moe_kernel,
        in_specs=[
            pl.BlockSpec(memory_space=pltpu.SMEM),
            pl.BlockSpec(memory_space=pltpu.MemorySpace.HBM),
            pl.BlockSpec(memory_space=pltpu.MemorySpace.HBM),
            pl.BlockSpec(memory_space=pltpu.VMEM),
        ],
        out_specs=[
            pl.BlockSpec(memory_space=pltpu.MemorySpace.HBM),
            pl.BlockSpec(memory_space=pltpu.SMEM),
        ],
        out_shape=[
            jax.ShapeDtypeStruct((B, S, D_OUT), jnp.float32),
            jax.ShapeDtypeStruct((B,), jnp.int32),
        ],
        scratch_shapes=[
            pltpu.VMEM((2, S, D_IN), jnp.float32),
            pltpu.VMEM((2, D_OUT, D_IN), jnp.float32),
            pltpu.VMEM((2, S, D_OUT), jnp.float32),
            pltpu.SMEM((B,), jnp.int32),
            pltpu.SemaphoreType.DMA((2,)),
            pltpu.SemaphoreType.DMA((2,)),
            pltpu.SemaphoreType.DMA((2,)),
        ],
    )(expert_probs, x, expert_weights, expert_biases)
    return (x_out, expert_idx)


# speedup vs baseline: 1.3237x; 1.0280x over previous
import jax
import jax.numpy as jnp
from jax import lax
from jax.experimental import pallas as pl
from jax.experimental.pallas import tpu as pltpu

B, S, D_IN, D_OUT, E = 4, 2048, 1024, 1024, 64

CH = 512
NC = S // CH
NBUF = 8
TOT = B * NC


def _moe_kernel(probs_ref, x_hbm, w_hbm, bias_ref,
                out_hbm, idx_ref,
                xv, wv, ov, idx_s,
                sem_x, sem_w, sem_o):
    def cp_x(c, slot):
        b, k = divmod(c, NC)
        return pltpu.make_async_copy(
            x_hbm.at[b, pl.ds(k * CH, CH), :], xv.at[slot], sem_x.at[slot])

    def cp_w(b):
        return pltpu.make_async_copy(w_hbm.at[idx_s[b]], wv.at[b],
                                     sem_w.at[b])

    def cp_o(c, slot):
        b, k = divmod(c, NC)
        return pltpu.make_async_copy(
            ov.at[slot], out_hbm.at[b, pl.ds(k * CH, CH), :], sem_o.at[slot])

    cp_x(0, 0).start()

    for b in range(B):
        def body(e, carry):
            best_v, best_i = carry
            v = probs_ref[b, e]
            better = v > best_v
            return (jnp.where(better, v, best_v),
                    jnp.where(better, e, best_i))
        _, best_i = lax.fori_loop(0, E, body,
                                  (probs_ref[b, 0], jnp.int32(0)))
        idx_s[b] = best_i
        idx_ref[b] = best_i

    cp_w(0).start()
    cp_x(1, 1).start()
    cp_w(1).start()
    cp_x(2, 2).start()
    cp_w(2).start()
    cp_x(3, 3).start()
    cp_w(3).start()
    for slot in range(4, NBUF):
        cp_x(slot, slot).start()

    for c in range(TOT):
        slot = c % NBUF
        b, k = divmod(c, NC)
        if k == 0:
            cp_w(b).wait()
        cp_x(c, slot).wait()
        if c >= NBUF:
            cp_o(c - NBUF, slot).wait()
        acc = lax.dot_general(
            xv[slot], wv[b],
            dimension_numbers=(((1,), (1,)), ((), ())),
            preferred_element_type=jnp.float32,
        )
        ov[slot] = acc + bias_ref[pl.ds(idx_s[b], 1), :]
        cp_o(c, slot).start()
        if c + NBUF < TOT:
            cp_x(c + NBUF, slot).start()

    for c in range(TOT - NBUF, TOT):
        cp_o(c, c % NBUF).wait()


def kernel(x, expert_probs, expert_weights, expert_biases):
    x_out, expert_idx = pl.pallas_call(
        _moe_kernel,
        in_specs=[
            pl.BlockSpec(memory_space=pltpu.SMEM),
            pl.BlockSpec(memory_space=pltpu.MemorySpace.HBM),
            pl.BlockSpec(memory_space=pltpu.MemorySpace.HBM),
            pl.BlockSpec(memory_space=pltpu.VMEM),
        ],
        out_specs=[
            pl.BlockSpec(memory_space=pltpu.MemorySpace.HBM),
            pl.BlockSpec(memory_space=pltpu.SMEM),
        ],
        out_shape=[
            jax.ShapeDtypeStruct((B, S, D_OUT), jnp.float32),
            jax.ShapeDtypeStruct((B,), jnp.int32),
        ],
        scratch_shapes=[
            pltpu.VMEM((NBUF, CH, D_IN), jnp.float32),
            pltpu.VMEM((B, D_OUT, D_IN), jnp.float32),
            pltpu.VMEM((NBUF, CH, D_OUT), jnp.float32),
            pltpu.SMEM((B,), jnp.int32),
            pltpu.SemaphoreType.DMA((NBUF,)),
            pltpu.SemaphoreType.DMA((B,)),
            pltpu.SemaphoreType.DMA((NBUF,)),
        ],
    )(expert_probs, x, expert_weights, expert_biases)
    return (x_out, expert_idx)


# speedup vs baseline: 1.4138x; 1.0681x over previous
import jax
import jax.numpy as jnp
from jax import lax
from jax.experimental import pallas as pl
from jax.experimental.pallas import tpu as pltpu

B, S, D_IN, D_OUT, E = 4, 2048, 1024, 1024, 64

CH = 1024
NC = S // CH
NBUF = 4
TOT = B * NC


def _moe_kernel(probs_ref, x_hbm, w_hbm, bias_ref,
                out_hbm, idx_ref,
                xv, wv, ov, idx_s,
                sem_x, sem_w, sem_o):
    def cp_x(c, slot):
        b, k = divmod(c, NC)
        return pltpu.make_async_copy(
            x_hbm.at[b, pl.ds(k * CH, CH), :], xv.at[slot], sem_x.at[slot])

    def cp_w(b):
        return pltpu.make_async_copy(w_hbm.at[idx_s[b]], wv.at[b],
                                     sem_w.at[b])

    def cp_o(c, slot):
        b, k = divmod(c, NC)
        return pltpu.make_async_copy(
            ov.at[slot], out_hbm.at[b, pl.ds(k * CH, CH), :], sem_o.at[slot])

    cp_x(0, 0).start()

    for b in range(B):
        def body(e, carry):
            best_v, best_i = carry
            v = probs_ref[b, e]
            better = v > best_v
            return (jnp.where(better, v, best_v),
                    jnp.where(better, e, best_i))
        _, best_i = lax.fori_loop(0, E, body,
                                  (probs_ref[b, 0], jnp.int32(0)))
        idx_s[b] = best_i
        idx_ref[b] = best_i

    cp_w(0).start()
    cp_x(1, 1).start()
    cp_w(1).start()
    cp_x(2, 2).start()
    cp_w(2).start()
    cp_x(3, 3).start()
    cp_w(3).start()
    for slot in range(4, NBUF):
        cp_x(slot, slot).start()

    for c in range(TOT):
        slot = c % NBUF
        b, k = divmod(c, NC)
        if k == 0:
            cp_w(b).wait()
        cp_x(c, slot).wait()
        if c >= NBUF:
            cp_o(c - NBUF, slot).wait()
        acc = lax.dot_general(
            xv[slot], wv[b],
            dimension_numbers=(((1,), (1,)), ((), ())),
            preferred_element_type=jnp.float32,
        )
        ov[slot] = acc + bias_ref[pl.ds(idx_s[b], 1), :]
        cp_o(c, slot).start()
        if c + NBUF < TOT:
            cp_x(c + NBUF, slot).start()

    for c in range(TOT - NBUF, TOT):
        cp_o(c, c % NBUF).wait()


def kernel(x, expert_probs, expert_weights, expert_biases):
    x_out, expert_idx = pl.pallas_call(
        _moe_kernel,
        in_specs=[
            pl.BlockSpec(memory_space=pltpu.SMEM),
            pl.BlockSpec(memory_space=pltpu.MemorySpace.HBM),
            pl.BlockSpec(memory_space=pltpu.MemorySpace.HBM),
            pl.BlockSpec(memory_space=pltpu.VMEM),
        ],
        out_specs=[
            pl.BlockSpec(memory_space=pltpu.MemorySpace.HBM),
            pl.BlockSpec(memory_space=pltpu.SMEM),
        ],
        out_shape=[
            jax.ShapeDtypeStruct((B, S, D_OUT), jnp.float32),
            jax.ShapeDtypeStruct((B,), jnp.int32),
        ],
        scratch_shapes=[
            pltpu.VMEM((NBUF, CH, D_IN), jnp.float32),
            pltpu.VMEM((B, D_OUT, D_IN), jnp.float32),
            pltpu.VMEM((NBUF, CH, D_OUT), jnp.float32),
            pltpu.SMEM((B,), jnp.int32),
            pltpu.SemaphoreType.DMA((NBUF,)),
            pltpu.SemaphoreType.DMA((B,)),
            pltpu.SemaphoreType.DMA((NBUF,)),
        ],
    )(expert_probs, x, expert_weights, expert_biases)
    return (x_out, expert_idx)
